# baseline (speedup 1.0000x reference)
import functools

import jax
import jax.numpy as jnp
from jax import lax
from jax.experimental import pallas as pl
from jax.experimental.pallas import tpu as pltpu
from jax.experimental.pallas import tpu_sc as plsc

N_NODES = 10000
N_EDGES = 160000
D = 256
DH = 128

SC_TILES = 16
K = 128
E_PER_TILE = N_EDGES // SC_TILES
NB = 4
CHUNKS_FULL = E_PER_TILE // K
GROUPS = 19
TAIL_OFF = CHUNKS_FULL * K
TAIL = E_PER_TILE - TAIL_OFF
N_RANGE = 5000
ACC_ROWS = 5376
ROWS_PER_TILE = ACC_ROWS // SC_TILES

A_BLOCK = 10000
C_BLOCK = 5000


def _elu(x):
    return jnp.where(x > 0, x, jnp.exp(jnp.minimum(x, 0.0)) - 1.0)


def _prep_body(x_ref, w0_ref, w1_ref, wf_ref, out_ref):
    h = _elu(jnp.dot(x_ref[...], w0_ref[...], preferred_element_type=jnp.float32))
    h = _elu(jnp.dot(h, w1_ref[...], preferred_element_type=jnp.float32))
    h = _elu(jnp.dot(h, wf_ref[...], preferred_element_type=jnp.float32))
    out_ref[0] = h[:, :DH]
    out_ref[1] = h[:, DH:]


def _prepare(x, w0, w1, wf):
    return pl.pallas_call(
        _prep_body,
        grid=(N_NODES // A_BLOCK,),
        in_specs=[
            pl.BlockSpec((A_BLOCK, D), lambda i: (i, 0)),
            pl.BlockSpec((D, D), lambda i: (0, 0)),
            pl.BlockSpec((D, D), lambda i: (0, 0)),
            pl.BlockSpec((D, D), lambda i: (0, 0)),
        ],
        out_specs=pl.BlockSpec((2, A_BLOCK, DH), lambda i: (0, i, 0)),
        out_shape=jax.ShapeDtypeStruct((2, N_NODES, DH), jnp.float32),
    )(x, w0, w1, wf)


def _sc_agg_body(src_hbm, dst_hbm, prep_hbm, zeros_hbm, out_hbm,
                 srcs_all, dsts_all,
                 srcb0, srcb1, srcb2, srcb3, srcb_t,
                 dstl0, dstl1, dstl2, dstl3, dstl_t,
                 rows0, rows1, rows2, rows3,
                 gsem0, gsem1, gsem2, gsem3,
                 ssem0, ssem1, ssem2, ssem3,
                 acc_sh):
    c = lax.axis_index("c")
    s = lax.axis_index("s")
    row0 = s * ROWS_PER_TILE
    lane = lax.iota(jnp.int32, 16)
    dummy = N_RANGE + s * 16 + lane
    srcbs = (srcb0, srcb1, srcb2, srcb3)
    dstls = (dstl0, dstl1, dstl2, dstl3)
    rows = (rows0, rows1, rows2, rows3)
    gsems = (gsem0, gsem1, gsem2, gsem3)
    ssems = (ssem0, ssem1, ssem2, ssem3)
    ebase = pl.multiple_of(s * E_PER_TILE, 8)

    pltpu.sync_copy(src_hbm.at[pl.ds(ebase, E_PER_TILE)], srcs_all)
    pltpu.sync_copy(dst_hbm.at[pl.ds(ebase, E_PER_TILE)], dsts_all)

    def localize_src(i, sref, n16):
        coff = c * N_NODES
        for g in range(n16):
            sref[pl.ds(g * 16, 16)] = srcs_all[pl.ds(i * K + g * 16, 16)] + coff

    def localize_dst(p, i, dref, n16):
        for g in range(n16):
            v = dsts_all[pl.ds(i * K + g * 16, 16)] - (p * N_RANGE)
            ok = (v >= 0) & (v < N_RANGE)
            dref[pl.ds(g * 16, 16)] = jnp.where(ok, v, dummy)

    def gather_desc(b):
        return pltpu.make_async_copy(prep_hbm.at[srcbs[b]], rows[b], gsems[b])

    def scatter_desc(b):
        return pltpu.make_async_copy(rows[b], acc_sh.at[dstls[b]], ssems[b])

    def start_gather(i, b):
        localize_src(i, srcbs[b], K // 16)
        pltpu.async_copy(prep_hbm.at[srcbs[b]], rows[b], gsems[b])

    def consume(p, i, b):
        gather_desc(b).wait()
        localize_dst(p, i, dstls[b], K // 16)
        pltpu.async_copy(rows[b], acc_sh.at[dstls[b]], ssems[b], add=True)

    for p in range(2):
        for b in range(NB):
            start_gather(b, b)

        pltpu.sync_copy(zeros_hbm, acc_sh.at[pl.ds(row0, ROWS_PER_TILE)])
        plsc.subcore_barrier()

        def group(j, carry):
            for b in range(NB):
                consume(p, j * NB + b, b)
            for b in range(NB):
                scatter_desc(b).wait()
                start_gather((j + 1) * NB + b, b)
            return carry

        lax.fori_loop(0, GROUPS - 1, group, 0)
        for b in range(NB):
            consume(p, (GROUPS - 1) * NB + b, b)
        for b in range(NB):
            scatter_desc(b).wait()

        for i in (GROUPS * NB, GROUPS * NB + 1):
            start_gather(i, 0)
            gather_desc(0).wait()
            localize_dst(p, i, dstl0, K // 16)
            pltpu.sync_copy(rows0, acc_sh.at[dstl0], add=True)
        localize_src(CHUNKS_FULL, srcb_t, TAIL // 16)
        pltpu.async_copy(prep_hbm.at[srcb_t],
                         rows0.at[pl.ds(0, TAIL)], gsem0)
        pltpu.make_async_copy(prep_hbm.at[srcb_t],
                              rows0.at[pl.ds(0, TAIL)], gsem0).wait()
        localize_dst(p, CHUNKS_FULL, dstl_t, TAIL // 16)
        pltpu.sync_copy(rows0.at[pl.ds(0, TAIL)], acc_sh.at[dstl_t], add=True)

        plsc.subcore_barrier()
        pltpu.sync_copy(acc_sh.at[pl.ds(row0, ROWS_PER_TILE)],
                        out_hbm.at[p, c, pl.ds(row0, ROWS_PER_TILE)])


def _sc_aggregate(src, dst, prep2, zeros_hbm):
    mesh = plsc.VectorSubcoreMesh(core_axis_name="c", subcore_axis_name="s")
    fn = functools.partial(
        pl.kernel,
        out_type=jax.ShapeDtypeStruct((2, 2, ACC_ROWS, DH), jnp.float32),
        mesh=mesh,
        scratch_types=(
            [pltpu.VMEM((E_PER_TILE,), jnp.int32)] * 2
            + [pltpu.VMEM((K,), jnp.int32)] * NB
            + [pltpu.VMEM((TAIL,), jnp.int32)]
            + [pltpu.VMEM((K,), jnp.int32)] * NB
            + [pltpu.VMEM((TAIL,), jnp.int32)]
            + [pltpu.VMEM((K, DH), jnp.float32)] * NB
            + [pltpu.SemaphoreType.DMA] * (2 * NB)
            + [pltpu.VMEM_SHARED((ACC_ROWS, DH), jnp.float32)]
        ),
    )(_sc_agg_body)
    return fn(src, dst, prep2, zeros_hbm)


def _upd_body(x_ref, agg_ref, wa_ref, wb_ref, bu0_ref, wu1_ref, bu1_ref,
              wuf_ref, buf_ref, o_ref):
    acc = jnp.dot(x_ref[...], wa_ref[...], preferred_element_type=jnp.float32)
    for h_ix in range(2):
        acc += jnp.dot(agg_ref[0, h_ix], wb_ref[h_ix],
                       preferred_element_type=jnp.float32)
    h = _elu(acc + bu0_ref[...])
    h = _elu(jnp.dot(h, wu1_ref[...], preferred_element_type=jnp.float32)
             + bu1_ref[...])
    y = jnp.dot(h, wuf_ref[...], preferred_element_type=jnp.float32) + buf_ref[...]
    ss = jnp.sum(y * y, axis=1, keepdims=True)
    o_ref[...] = y * lax.rsqrt(jnp.maximum(ss, 1e-12))


def _update(x, agg_full, wa, wb, bu0, wu1, bu1, wuf, buf_):
    grid = N_NODES // C_BLOCK
    nb = N_RANGE // C_BLOCK
    full = lambda shape: pl.BlockSpec(shape, lambda i, _s=shape: tuple(0 for _ in _s))
    return pl.pallas_call(
        _upd_body,
        grid=(grid,),
        in_specs=[
            pl.BlockSpec((C_BLOCK, D), lambda i: (i, 0)),
            pl.BlockSpec((1, 2, C_BLOCK, DH), lambda i: (i // nb, 0, i % nb, 0)),
            full((D, D)),
            full((2, DH, D)),
            full((1, D)),
            full((D, D)),
            full((1, D)),
            full((D, D)),
            full((1, D)),
        ],
        out_specs=pl.BlockSpec((C_BLOCK, D), lambda i: (i, 0)),
        out_shape=jax.ShapeDtypeStruct((N_NODES, D), jnp.float32),
    )(x, agg_full, wa, wb, bu0, wu1, bu1, wuf, buf_)


def kernel(node_representations, edges, edge_weights,
           W_p0, W_p1, W_pf, W_u0, b_u0, W_u1, b_u1, W_uf, b_uf):
    del edge_weights
    x = node_representations[0]
    edges_t = edges.T
    src = edges_t[0]
    dst = edges_t[1]

    prep = _prepare(x, W_p0, W_p1, W_pf)
    prep2 = prep.reshape(2 * N_NODES, DH)

    zeros_hbm = jnp.zeros((ROWS_PER_TILE, DH), jnp.float32)
    agg_full = _sc_aggregate(src, dst, prep2, zeros_hbm)

    wa = W_u0[:D]
    wb = W_u0[D:].reshape(2, DH, D)
    y = _update(x, agg_full, wa, wb, b_u0.reshape(1, D), W_u1,
                b_u1.reshape(1, D), W_uf, b_uf.reshape(1, D))
    return y.reshape(1, N_NODES, D)


# speedup vs baseline: 1.0030x; 1.0030x over previous
import functools

import jax
import jax.numpy as jnp
from jax import lax
from jax.experimental import pallas as pl
from jax.experimental.pallas import tpu as pltpu
from jax.experimental.pallas import tpu_sc as plsc

N_NODES = 10000
N_EDGES = 160000
D = 256
DH = 128

SC_TILES = 16
K = 128
E_PER_TILE = N_EDGES // SC_TILES
NB = 4
CHUNKS_FULL = E_PER_TILE // K
GROUPS = 19
TAIL_OFF = CHUNKS_FULL * K
TAIL = E_PER_TILE - TAIL_OFF
N_RANGE = 5000
ACC_ROWS = 5376
ROWS_PER_TILE = ACC_ROWS // SC_TILES

A_BLOCK = 2000
C_BLOCK = 1000


def _elu(x):
    return jnp.where(x > 0, x, jnp.exp(jnp.minimum(x, 0.0)) - 1.0)


def _prep_body(x_ref, w0_ref, w1_ref, wf_ref, out_ref):
    h = _elu(jnp.dot(x_ref[...], w0_ref[...], preferred_element_type=jnp.float32))
    h = _elu(jnp.dot(h, w1_ref[...], preferred_element_type=jnp.float32))
    h = _elu(jnp.dot(h, wf_ref[...], preferred_element_type=jnp.float32))
    out_ref[0] = h[:, :DH]
    out_ref[1] = h[:, DH:]


def _prepare(x, w0, w1, wf):
    return pl.pallas_call(
        _prep_body,
        grid=(N_NODES // A_BLOCK,),
        in_specs=[
            pl.BlockSpec((A_BLOCK, D), lambda i: (i, 0)),
            pl.BlockSpec((D, D), lambda i: (0, 0)),
            pl.BlockSpec((D, D), lambda i: (0, 0)),
            pl.BlockSpec((D, D), lambda i: (0, 0)),
        ],
        out_specs=pl.BlockSpec((2, A_BLOCK, DH), lambda i: (0, i, 0)),
        out_shape=jax.ShapeDtypeStruct((2, N_NODES, DH), jnp.float32),
    )(x, w0, w1, wf)


def _sc_agg_body(src_hbm, dst_hbm, prep_hbm, zeros_hbm, out_hbm,
                 srcs_all, dsts_all,
                 srcb0, srcb1, srcb2, srcb3, srcb_t,
                 dstl0, dstl1, dstl2, dstl3, dstl_t,
                 rows0, rows1, rows2, rows3,
                 gsem0, gsem1, gsem2, gsem3,
                 ssem0, ssem1, ssem2, ssem3,
                 acc_sh):
    c = lax.axis_index("c")
    s = lax.axis_index("s")
    row0 = s * ROWS_PER_TILE
    lane = lax.iota(jnp.int32, 16)
    dummy = N_RANGE + s * 16 + lane
    srcbs = (srcb0, srcb1, srcb2, srcb3)
    dstls = (dstl0, dstl1, dstl2, dstl3)
    rows = (rows0, rows1, rows2, rows3)
    gsems = (gsem0, gsem1, gsem2, gsem3)
    ssems = (ssem0, ssem1, ssem2, ssem3)
    ebase = pl.multiple_of(s * E_PER_TILE, 8)

    pltpu.sync_copy(src_hbm.at[pl.ds(ebase, E_PER_TILE)], srcs_all)
    pltpu.sync_copy(dst_hbm.at[pl.ds(ebase, E_PER_TILE)], dsts_all)

    def localize_src(i, sref, n16):
        coff = c * N_NODES
        for g in range(n16):
            sref[pl.ds(g * 16, 16)] = srcs_all[pl.ds(i * K + g * 16, 16)] + coff

    def localize_dst(p, i, dref, n16):
        for g in range(n16):
            v = dsts_all[pl.ds(i * K + g * 16, 16)] - (p * N_RANGE)
            ok = (v >= 0) & (v < N_RANGE)
            dref[pl.ds(g * 16, 16)] = jnp.where(ok, v, dummy)

    def gather_desc(b):
        return pltpu.make_async_copy(prep_hbm.at[srcbs[b]], rows[b], gsems[b])

    def scatter_desc(b):
        return pltpu.make_async_copy(rows[b], acc_sh.at[dstls[b]], ssems[b])

    def start_gather(i, b):
        localize_src(i, srcbs[b], K // 16)
        pltpu.async_copy(prep_hbm.at[srcbs[b]], rows[b], gsems[b])

    def consume(p, i, b):
        gather_desc(b).wait()
        localize_dst(p, i, dstls[b], K // 16)
        pltpu.async_copy(rows[b], acc_sh.at[dstls[b]], ssems[b], add=True)

    for p in range(2):
        for b in range(NB):
            start_gather(b, b)

        pltpu.sync_copy(zeros_hbm, acc_sh.at[pl.ds(row0, ROWS_PER_TILE)])
        plsc.subcore_barrier()

        def group(j, carry):
            for b in range(NB):
                consume(p, j * NB + b, b)
            for b in range(NB):
                scatter_desc(b).wait()
                start_gather((j + 1) * NB + b, b)
            return carry

        lax.fori_loop(0, GROUPS - 1, group, 0)
        for b in range(NB):
            consume(p, (GROUPS - 1) * NB + b, b)
        for b in range(NB):
            scatter_desc(b).wait()

        for i in (GROUPS * NB, GROUPS * NB + 1):
            start_gather(i, 0)
            gather_desc(0).wait()
            localize_dst(p, i, dstl0, K // 16)
            pltpu.sync_copy(rows0, acc_sh.at[dstl0], add=True)
        localize_src(CHUNKS_FULL, srcb_t, TAIL // 16)
        pltpu.async_copy(prep_hbm.at[srcb_t],
                         rows0.at[pl.ds(0, TAIL)], gsem0)
        pltpu.make_async_copy(prep_hbm.at[srcb_t],
                              rows0.at[pl.ds(0, TAIL)], gsem0).wait()
        localize_dst(p, CHUNKS_FULL, dstl_t, TAIL // 16)
        pltpu.sync_copy(rows0.at[pl.ds(0, TAIL)], acc_sh.at[dstl_t], add=True)

        plsc.subcore_barrier()
        pltpu.sync_copy(acc_sh.at[pl.ds(row0, ROWS_PER_TILE)],
                        out_hbm.at[p, c, pl.ds(row0, ROWS_PER_TILE)])


def _sc_aggregate(src, dst, prep2, zeros_hbm):
    mesh = plsc.VectorSubcoreMesh(core_axis_name="c", subcore_axis_name="s")
    fn = functools.partial(
        pl.kernel,
        out_type=jax.ShapeDtypeStruct((2, 2, ACC_ROWS, DH), jnp.float32),
        mesh=mesh,
        scratch_types=(
            [pltpu.VMEM((E_PER_TILE,), jnp.int32)] * 2
            + [pltpu.VMEM((K,), jnp.int32)] * NB
            + [pltpu.VMEM((TAIL,), jnp.int32)]
            + [pltpu.VMEM((K,), jnp.int32)] * NB
            + [pltpu.VMEM((TAIL,), jnp.int32)]
            + [pltpu.VMEM((K, DH), jnp.float32)] * NB
            + [pltpu.SemaphoreType.DMA] * (2 * NB)
            + [pltpu.VMEM_SHARED((ACC_ROWS, DH), jnp.float32)]
        ),
    )(_sc_agg_body)
    return fn(src, dst, prep2, zeros_hbm)


def _upd_body(x_ref, agg_ref, wa_ref, wb_ref, bu0_ref, wu1_ref, bu1_ref,
              wuf_ref, buf_ref, o_ref):
    acc = jnp.dot(x_ref[...], wa_ref[...], preferred_element_type=jnp.float32)
    for h_ix in range(2):
        acc += jnp.dot(agg_ref[0, h_ix], wb_ref[h_ix],
                       preferred_element_type=jnp.float32)
    h = _elu(acc + bu0_ref[...])
    h = _elu(jnp.dot(h, wu1_ref[...], preferred_element_type=jnp.float32)
             + bu1_ref[...])
    y = jnp.dot(h, wuf_ref[...], preferred_element_type=jnp.float32) + buf_ref[...]
    ss = jnp.sum(y * y, axis=1, keepdims=True)
    o_ref[...] = y * lax.rsqrt(jnp.maximum(ss, 1e-12))


def _update(x, agg_full, wa, wb, bu0, wu1, bu1, wuf, buf_):
    grid = N_NODES // C_BLOCK
    nb = N_RANGE // C_BLOCK
    full = lambda shape: pl.BlockSpec(shape, lambda i, _s=shape: tuple(0 for _ in _s))
    return pl.pallas_call(
        _upd_body,
        grid=(grid,),
        in_specs=[
            pl.BlockSpec((C_BLOCK, D), lambda i: (i, 0)),
            pl.BlockSpec((1, 2, C_BLOCK, DH), lambda i: (i // nb, 0, i % nb, 0)),
            full((D, D)),
            full((2, DH, D)),
            full((1, D)),
            full((D, D)),
            full((1, D)),
            full((D, D)),
            full((1, D)),
        ],
        out_specs=pl.BlockSpec((C_BLOCK, D), lambda i: (i, 0)),
        out_shape=jax.ShapeDtypeStruct((N_NODES, D), jnp.float32),
    )(x, agg_full, wa, wb, bu0, wu1, bu1, wuf, buf_)


def kernel(node_representations, edges, edge_weights,
           W_p0, W_p1, W_pf, W_u0, b_u0, W_u1, b_u1, W_uf, b_uf):
    del edge_weights
    x = node_representations[0]
    edges_t = edges.T
    src = edges_t[0]
    dst = edges_t[1]

    prep = _prepare(x, W_p0, W_p1, W_pf)
    prep2 = prep.reshape(2 * N_NODES, DH)

    zeros_hbm = jnp.zeros((ROWS_PER_TILE, DH), jnp.float32)
    agg_full = _sc_aggregate(src, dst, prep2, zeros_hbm)

    wa = W_u0[:D]
    wb = W_u0[D:].reshape(2, DH, D)
    y = _update(x, agg_full, wa, wb, b_u0.reshape(1, D), W_u1,
                b_u1.reshape(1, D), W_uf, b_uf.reshape(1, D))
    return y.reshape(1, N_NODES, D)
